# baseline (speedup 1.0000x reference)
import functools

import jax
import jax.numpy as jnp
from jax import lax
from jax.experimental import pallas as pl
from jax.experimental.pallas import tpu as pltpu
from jax.experimental.pallas import tpu_sc as plsc

B = 16384
F = 26
V = 100000
E = 64

NC = 2
NS = 16
NW = NC * NS

ROWS = B * F
IDX_ROWS = ROWS // 128
IDX_PER_W = IDX_ROWS // NW


def _sc_gather_body(table_hbm, idx_hbm, out_hbm, idx_v, rows_v, sem0, sem1):
    wid = lax.axis_index("s") * NC + lax.axis_index("c")
    row_base = wid * IDX_PER_W
    pltpu.sync_copy(idx_hbm.at[pl.ds(row_base, IDX_PER_W)], idx_v)

    def start(j, buf, sem):
        pltpu.async_copy(table_hbm.at[idx_v.at[j]], rows_v.at[buf], sem)

    def drain_write(j, buf, sem):
        pltpu.make_async_copy(table_hbm.at[idx_v.at[j]], rows_v.at[buf], sem).wait()
        pltpu.sync_copy(rows_v.at[buf], out_hbm.at[pl.ds((row_base + j) * 128, 128)])

    start(0, 0, sem0)

    def step(j, _):
        buf = lax.rem(j, 2)

        @pl.when(j + 1 < IDX_PER_W)
        def _():
            lax.cond(buf == 0,
                     lambda: start(j + 1, 1, sem1),
                     lambda: start(j + 1, 0, sem0))
        lax.cond(buf == 0,
                 lambda: drain_write(j, 0, sem0),
                 lambda: drain_write(j, 1, sem1))
        return 0

    lax.fori_loop(0, IDX_PER_W, step, 0)


def _sc_gather(tables2, idx2d):
    mesh = plsc.VectorSubcoreMesh(core_axis_name="c", subcore_axis_name="s",
                                  num_cores=NC, num_subcores=NS)
    return pl.kernel(
        _sc_gather_body,
        out_type=jax.ShapeDtypeStruct((ROWS, E), jnp.float32),
        mesh=mesh,
        compiler_params=pltpu.CompilerParams(use_tc_tiling_on_sc=False),
        scratch_types=[
            pltpu.VMEM((IDX_PER_W, 128), jnp.int32),
            pltpu.VMEM((2, 128, E), jnp.float32),
            pltpu.SemaphoreType.DMA,
            pltpu.SemaphoreType.DMA,
        ],
    )(tables2, idx2d)


BT = 512


def _tc_body(g_ref, xn_ref, wc_ref, wn_ref, bn_ref, wf_ref, bf_ref, out_ref):
    num = jnp.dot(xn_ref[...], wn_ref[...],
                  preferred_element_type=jnp.float32) + bn_ref[...]
    acc = jnp.dot(g_ref[...], wc_ref[...], preferred_element_type=jnp.float32)
    acc += jnp.dot(num, wf_ref[...], preferred_element_type=jnp.float32)
    out_ref[...] = acc + bf_ref[...]


def _tc_matmul(g, X_num, W_cat, W_num, b_num, W_fnum, b_final):
    grid = (B // BT,)
    return pl.pallas_call(
        _tc_body,
        grid=grid,
        in_specs=[
            pl.BlockSpec((BT, F * E), lambda i: (i, 0)),
            pl.BlockSpec((BT, X_num.shape[1]), lambda i: (i, 0)),
            pl.BlockSpec((F * E, E), lambda i: (0, 0)),
            pl.BlockSpec((X_num.shape[1], E), lambda i: (0, 0)),
            pl.BlockSpec((1, E), lambda i: (0, 0)),
            pl.BlockSpec((E, E), lambda i: (0, 0)),
            pl.BlockSpec((1, E), lambda i: (0, 0)),
        ],
        out_specs=pl.BlockSpec((BT, E), lambda i: (i, 0)),
        out_shape=jax.ShapeDtypeStruct((B, E), jnp.float32),
    )(g, X_num, W_cat, W_num, b_num, W_fnum, b_final)


def kernel(X_cat, X_num, tables, W_num, b_num, W_final, b_final):
    tables2 = tables.reshape(F * V, E)
    idx_flat = (X_cat.astype(jnp.int32)
                + (jnp.arange(F, dtype=jnp.int32) * V)[None, :])
    idx2d = idx_flat.reshape(IDX_ROWS, 128)
    g = _sc_gather(tables2, idx2d).reshape(B, F * E)
    W_cat = W_final[:F * E]
    W_fnum = W_final[F * E:]
    out = _tc_matmul(g, X_num, W_cat, W_num,
                     b_num.reshape(1, E), W_fnum, b_final.reshape(1, E))
    return out
